# Initial kernel scaffold; baseline (speedup 1.0000x reference)
import jax
import jax.numpy as jnp
from jax.experimental import pallas as pl


def kernel(token_ids, table):
    raise NotImplementedError("write your pallas kernel here")


# speedup vs baseline: 1.4759x; 1.4759x over previous
import functools

import jax
import jax.numpy as jnp
from jax import lax
from jax.experimental import pallas as pl
from jax.experimental.pallas import tpu as pltpu
from jax.experimental.pallas import tpu_sc as plsc


def _make_sc_embed(B_total, D, NC, NS, chunk):
    NW = NC * NS
    b_per_w = B_total // NW
    n_chunks = b_per_w // chunk
    mesh = plsc.VectorSubcoreMesh(core_axis_name="c", subcore_axis_name="s")

    @functools.partial(
        pl.kernel,
        mesh=mesh,
        out_type=jax.ShapeDtypeStruct((B_total, D), jnp.float32),
        compiler_params=pltpu.CompilerParams(use_tc_tiling_on_sc=False),
        scratch_types=[
            pltpu.VMEM((chunk,), jnp.int32),
            pltpu.VMEM((chunk, D), jnp.float32),
            pltpu.SemaphoreType.DMA,
        ],
    )
    def emb(idx_hbm, table_hbm, out_hbm, idx_v, rows_v, sem):
        wid = lax.axis_index("s") * NC + lax.axis_index("c")
        base = wid * b_per_w
        for j in range(n_chunks):
            off = base + j * chunk
            pltpu.sync_copy(idx_hbm.at[pl.ds(off, chunk)], idx_v)
            pltpu.async_copy(table_hbm.at[idx_v], rows_v, sem).wait()
            pltpu.sync_copy(rows_v, out_hbm.at[pl.ds(off, chunk)])

    return emb


def kernel(token_ids, table):
    B, S = token_ids.shape
    V, D = table.shape
    B_total = B * S
    idx = token_ids.reshape(B_total).astype(jnp.int32)
    info = plsc.get_sparse_core_info()
    NC, NS = info.num_cores, info.num_subcores
    emb = _make_sc_embed(B_total, D, NC, NS, chunk=1600)
    out = emb(idx, table)
    return out.reshape(B, S, D)


# speedup vs baseline: 1.5012x; 1.0171x over previous
import functools

import jax
import jax.numpy as jnp
from jax import lax
from jax.experimental import pallas as pl
from jax.experimental.pallas import tpu as pltpu
from jax.experimental.pallas import tpu_sc as plsc


def _make_sc_embed(B_total, D, NC, NS, chunk):
    NW = NC * NS
    b_per_w = B_total // NW
    n_chunks = b_per_w // chunk
    mesh = plsc.VectorSubcoreMesh(core_axis_name="c", subcore_axis_name="s")

    @functools.partial(
        pl.kernel,
        mesh=mesh,
        out_type=jax.ShapeDtypeStruct((B_total, D), jnp.float32),
        compiler_params=pltpu.CompilerParams(use_tc_tiling_on_sc=False),
        scratch_types=[
            pltpu.VMEM((b_per_w,), jnp.int32),
            pltpu.VMEM((chunk, D), jnp.float32),
            pltpu.VMEM((chunk, D), jnp.float32),
            pltpu.SemaphoreType.DMA,
            pltpu.SemaphoreType.DMA,
            pltpu.SemaphoreType.DMA,
            pltpu.SemaphoreType.DMA,
        ],
    )
    def emb(idx_hbm, table_hbm, out_hbm, idx_v, rows0, rows1, g0, g1, o0, o1):
        wid = lax.axis_index("s") * NC + lax.axis_index("c")
        base = wid * b_per_w
        pltpu.sync_copy(idx_hbm.at[pl.ds(base, b_per_w)], idx_v)

        rows = [rows0, rows1]
        gsem = [g0, g1]
        osem = [o0, o1]

        def start_gather(j):
            b = j % 2
            return pltpu.async_copy(
                table_hbm.at[idx_v.at[pl.ds(j * chunk, chunk)]], rows[b], gsem[b]
            )

        def start_store(j):
            b = j % 2
            return pltpu.async_copy(
                rows[b], out_hbm.at[pl.ds(base + j * chunk, chunk)], osem[b]
            )

        g_h = [None] * n_chunks
        o_h = [None] * n_chunks
        g_h[0] = start_gather(0)
        for j in range(n_chunks):
            if j + 1 < n_chunks:
                if j >= 1:
                    o_h[j - 1].wait()
                g_h[j + 1] = start_gather(j + 1)
            g_h[j].wait()
            o_h[j] = start_store(j)
        if n_chunks >= 2:
            o_h[n_chunks - 2].wait()
        o_h[n_chunks - 1].wait()

    return emb


def kernel(token_ids, table):
    B, S = token_ids.shape
    V, D = table.shape
    B_total = B * S
    idx = token_ids.reshape(B_total).astype(jnp.int32)
    info = plsc.get_sparse_core_info()
    NC, NS = info.num_cores, info.num_subcores
    emb = _make_sc_embed(B_total, D, NC, NS, chunk=1600)
    out = emb(idx, table)
    return out.reshape(B, S, D)
